# baseline (speedup 1.0000x reference)
import functools

import jax
import jax.numpy as jnp
from jax import lax
from jax.experimental import pallas as pl
from jax.experimental.pallas import tpu as pltpu
from jax.experimental.pallas import tpu_sc as plsc

NUM_EMB = 1_000_000
DIM = 32
BATCH = 16384

_NC = 2
_NS = 16
_NW = _NC * _NS
_BPW = BATCH // _NW
_CHUNK = 128
_NCH = _BPW // _CHUNK

_mesh = plsc.VectorSubcoreMesh(core_axis_name="c", subcore_axis_name="s")


@functools.partial(
    pl.kernel,
    mesh=_mesh,
    out_type=jax.ShapeDtypeStruct((BATCH, DIM), jnp.float32),
    scratch_types=[
        pltpu.VMEM((_NCH, _CHUNK), jnp.int32),
        pltpu.VMEM((_BPW, DIM), jnp.float32),
        pltpu.SemaphoreType.DMA,
    ],
    compiler_params=pltpu.CompilerParams(use_tc_tiling_on_sc=False),
)
def _emb_lookup(idx_hbm, table_hbm, out_hbm, idx_v, rows_v, sem):
    wid = lax.axis_index("s") * _NC + lax.axis_index("c")
    base = wid * _BPW
    pltpu.sync_copy(idx_hbm.at[wid], idx_v)
    copies = [
        pltpu.async_copy(
            table_hbm.at[idx_v.at[j]],
            rows_v.at[pl.ds(j * _CHUNK, _CHUNK)],
            sem,
        )
        for j in range(_NCH)
    ]
    for c in copies:
        c.wait()
    pltpu.sync_copy(rows_v, out_hbm.at[pl.ds(base, _BPW)])


def kernel(x, weight):
    idx = x.astype(jnp.int32).reshape(_NW, _NCH, _CHUNK)
    return _emb_lookup(idx, weight)


# speedup vs baseline: 1.3505x; 1.3505x over previous
import functools

import jax
import jax.numpy as jnp
from jax import lax
from jax.experimental import pallas as pl
from jax.experimental.pallas import tpu as pltpu
from jax.experimental.pallas import tpu_sc as plsc

NUM_EMB = 1_000_000
DIM = 32
BATCH = 16384

_NC = 2
_NS = 16
_NW = _NC * _NS
_BPW = BATCH // _NW
_CHUNK = 128
_NCH = _BPW // _CHUNK
_L = 16

_mesh = plsc.VectorSubcoreMesh(core_axis_name="c", subcore_axis_name="s")


@functools.partial(
    pl.kernel,
    mesh=_mesh,
    out_type=jax.ShapeDtypeStruct((DIM, BATCH), jnp.float32),
    scratch_types=[
        pltpu.VMEM((_BPW + _L,), jnp.int32),
        pltpu.VMEM((_CHUNK // 8, 4, 8, 128), jnp.float32),
        pltpu.VMEM((DIM, _CHUNK), jnp.float32),
        pltpu.SemaphoreType.DMA,
    ],
    compiler_params=pltpu.CompilerParams(
        use_tc_tiling_on_sc=True, needs_layout_passes=False
    ),
)
def _emb_lookup(idx_hbm, wt_hbm, out_hbm, xv, gbuf, dst, sem):
    wid = lax.axis_index("s") * _NC + lax.axis_index("c")
    base = wid * _BPW
    pltpu.sync_copy(idx_hbm.at[pl.ds(base, _BPW)], xv.at[pl.ds(0, _BPW)])
    iota = lax.iota(jnp.int32, _L)

    def chunk_body(c):
        cbase = c * _CHUNK

        def group_fire(h):
            vec = xv[pl.ds(cbase + h * 8, _L)]
            vb = (vec >> 7) << 7
            rr = (vec >> 4) & 7
            for jj in range(8):
                tile_slice = wt_hbm.at[
                    :, :, pl.ds(pl.multiple_of(vb[jj], 128), 128)
                ]
                dst_slice = gbuf.at[h, :, :, pl.ds(jj * _L, _L)]
                for r in range(8):
                    @pl.when(rr[jj] == r)
                    def _fire(ts=tile_slice, ds_=dst_slice, r=r):
                        pltpu.async_copy(
                            ts.at[:, :, pl.ds(r * _L, _L)], ds_, sem
                        )
            for jj in range(8):
                pltpu.make_async_copy(
                    wt_hbm.at[:, :, pl.ds(0, _L)],
                    gbuf.at[h, :, :, pl.ds(jj * _L, _L)],
                    sem,
                ).wait()

        pl.loop(0, _CHUNK // 8)(group_fire)

        def group_extract(g):
            vec = xv[pl.ds(cbase + g * _L, _L)]
            blk = g * 2 + (iota >> 3)
            lane = ((iota & 7) << 4) + (vec & 15)
            for d in range(DIM):
                tr, s = d // 8, d % 8
                dst[d, pl.ds(g * _L, _L)] = plsc.load_gather(
                    gbuf, [blk, jnp.full((_L,), tr, jnp.int32),
                           jnp.full((_L,), s, jnp.int32), lane]
                )

        pl.loop(0, _CHUNK // _L)(group_extract)

        for tr in range(4):
            pltpu.sync_copy(
                dst.at[pl.ds(tr * 8, 8)],
                out_hbm.at[pl.ds(tr * 8, 8), pl.ds(base + cbase, _CHUNK)],
            )

    pl.loop(0, _NCH)(chunk_body)


def kernel(x, weight):
    out_t = _emb_lookup(x.astype(jnp.int32), weight.T.reshape(4, 8, NUM_EMB))
    return out_t.T


# speedup vs baseline: 1.3518x; 1.0009x over previous
import functools

import jax
import jax.numpy as jnp
from jax import lax
from jax.experimental import pallas as pl
from jax.experimental.pallas import tpu as pltpu
from jax.experimental.pallas import tpu_sc as plsc

NUM_EMB = 1_000_000
DIM = 32
BATCH = 16384

_NC = 2
_NS = 16
_NW = _NC * _NS
_BPW = BATCH // _NW
_CHUNK = 128
_NCH = _BPW // _CHUNK
_L = 16

_mesh = plsc.VectorSubcoreMesh(core_axis_name="c", subcore_axis_name="s")


@functools.partial(
    pl.kernel,
    mesh=_mesh,
    out_type=jax.ShapeDtypeStruct((DIM, BATCH), jnp.float32),
    scratch_types=[
        pltpu.VMEM((_BPW + _L,), jnp.int32),
        pltpu.VMEM((_CHUNK // 8, 4, 8, 128), jnp.float32),
        pltpu.VMEM((DIM, _CHUNK), jnp.float32),
        pltpu.SemaphoreType.DMA,
    ],
    compiler_params=pltpu.CompilerParams(
        use_tc_tiling_on_sc=True, needs_layout_passes=False
    ),
)
def _emb_lookup(idx_hbm, wt_hbm, out_hbm, xv, gbuf, dst, sem):
    wid = lax.axis_index("s") * _NC + lax.axis_index("c")
    base = wid * _BPW
    pltpu.sync_copy(idx_hbm.at[pl.ds(base, _BPW)], xv.at[pl.ds(0, _BPW)])
    iota = lax.iota(jnp.int32, _L)

    def chunk_body(c):
        cbase = c * _CHUNK

        def group_fire(h):
            @pl.when(h > 0)
            def _drain_prev():
                for jj in range(8):
                    pltpu.make_async_copy(
                        wt_hbm.at[:, :, pl.ds(0, _L)],
                        gbuf.at[h - 1, :, :, pl.ds(jj * _L, _L)],
                        sem,
                    ).wait()

            vec = xv[pl.ds(cbase + h * 8, _L)]
            vb = (vec >> 7) << 7
            rr = (vec >> 4) & 7
            for jj in range(8):
                tile_slice = wt_hbm.at[
                    :, :, pl.ds(pl.multiple_of(vb[jj], 128), 128)
                ]
                dst_slice = gbuf.at[h, :, :, pl.ds(jj * _L, _L)]
                for r in range(8):
                    @pl.when(rr[jj] == r)
                    def _fire(ts=tile_slice, ds_=dst_slice, r=r):
                        pltpu.async_copy(
                            ts.at[:, :, pl.ds(r * _L, _L)], ds_, sem
                        )

        pl.loop(0, _CHUNK // 8)(group_fire)
        for jj in range(8):
            pltpu.make_async_copy(
                wt_hbm.at[:, :, pl.ds(0, _L)],
                gbuf.at[_CHUNK // 8 - 1, :, :, pl.ds(jj * _L, _L)],
                sem,
            ).wait()

        def group_extract(g):
            vec = xv[pl.ds(cbase + g * _L, _L)]
            blk = g * 2 + (iota >> 3)
            lane = ((iota & 7) << 4) + (vec & 15)
            for d in range(DIM):
                tr, s = d // 8, d % 8
                dst[d, pl.ds(g * _L, _L)] = plsc.load_gather(
                    gbuf, [blk, jnp.full((_L,), tr, jnp.int32),
                           jnp.full((_L,), s, jnp.int32), lane]
                )

        pl.loop(0, _CHUNK // _L)(group_extract)

        for tr in range(4):
            pltpu.sync_copy(
                dst.at[pl.ds(tr * 8, 8)],
                out_hbm.at[pl.ds(tr * 8, 8), pl.ds(base + cbase, _CHUNK)],
            )

    pl.loop(0, _NCH)(chunk_body)


def kernel(x, weight):
    out_t = _emb_lookup(x.astype(jnp.int32), weight.T.reshape(4, 8, NUM_EMB))
    return out_t.T


# speedup vs baseline: 3.8641x; 2.8586x over previous
import functools

import jax
import jax.numpy as jnp
from jax import lax
from jax.experimental import pallas as pl
from jax.experimental.pallas import tpu as pltpu
from jax.experimental.pallas import tpu_sc as plsc

NUM_EMB = 1_000_000
DIM = 32
BATCH = 16384

_NC = 2
_NS = 16
_NW = _NC * _NS
_BPW = BATCH // _NW
_CHUNK = 128
_NCH = _BPW // _CHUNK
_L = 16

_mesh = plsc.VectorSubcoreMesh(core_axis_name="c", subcore_axis_name="s")


@functools.partial(
    pl.kernel,
    mesh=_mesh,
    out_type=jax.ShapeDtypeStruct((DIM, BATCH), jnp.float32),
    scratch_types=[
        pltpu.VMEM((_BPW,), jnp.int32),
        pltpu.VMEM((_L, 4, 8, 128), jnp.float32),
        pltpu.VMEM((DIM, _CHUNK), jnp.float32),
        pltpu.SemaphoreType.DMA,
    ],
    compiler_params=pltpu.CompilerParams(
        use_tc_tiling_on_sc=True, needs_layout_passes=False
    ),
)
def _emb_lookup(idx_hbm, wt_hbm, out_hbm, xv, gbuf, dst, sem):
    wid = lax.axis_index("s") * _NC + lax.axis_index("c")
    base = wid * _BPW
    pltpu.sync_copy(idx_hbm.at[pl.ds(base, _BPW)], xv)
    iota = lax.iota(jnp.int32, _L)

    def chunk_body(c):
        cbase = c * _CHUNK

        def round_body(g):
            vec = xv[pl.ds(cbase + g * _L, _L)]
            vb = (vec >> 7) << 7
            copies = [
                pltpu.async_copy(
                    wt_hbm.at[:, :, pl.ds(pl.multiple_of(vb[j], 128), 128)],
                    gbuf.at[j],
                    sem,
                )
                for j in range(_L)
            ]
            for cp in copies:
                cp.wait()
            lane = vec & 127
            for d in range(DIM):
                tr, s = d // 8, d % 8
                dst[d, pl.ds(g * _L, _L)] = plsc.load_gather(
                    gbuf, [iota, jnp.full((_L,), tr, jnp.int32),
                           jnp.full((_L,), s, jnp.int32), lane]
                )

        pl.loop(0, _CHUNK // _L)(round_body)

        for tr in range(4):
            pltpu.sync_copy(
                dst.at[pl.ds(tr * 8, 8)],
                out_hbm.at[pl.ds(tr * 8, 8), pl.ds(base + cbase, _CHUNK)],
            )

    pl.loop(0, _NCH)(chunk_body)


def kernel(x, weight):
    out_t = _emb_lookup(x.astype(jnp.int32), weight.T.reshape(4, 8, NUM_EMB))
    return out_t.T


# speedup vs baseline: 3.8694x; 1.0014x over previous
import functools

import jax
import jax.numpy as jnp
from jax import lax
from jax.experimental import pallas as pl
from jax.experimental.pallas import tpu as pltpu
from jax.experimental.pallas import tpu_sc as plsc

NUM_EMB = 1_000_000
DIM = 32
BATCH = 16384

_NC = 2
_NS = 16
_NW = _NC * _NS
_BPW = BATCH // _NW
_CHUNK = 128
_NCH = _BPW // _CHUNK
_L = 16
_BKT = _CHUNK + _L

_mesh = plsc.VectorSubcoreMesh(core_axis_name="c", subcore_axis_name="s")


@functools.partial(
    pl.kernel,
    mesh=_mesh,
    out_type=jax.ShapeDtypeStruct((DIM, BATCH), jnp.float32),
    scratch_types=[
        pltpu.VMEM((_BPW,), jnp.int32),
        pltpu.VMEM((8, _BKT), jnp.int32),
        pltpu.VMEM((8, _BKT), jnp.int32),
        pltpu.VMEM((_L, 4, 8, 128), jnp.float32),
        pltpu.VMEM((DIM, _CHUNK + _L), jnp.float32),
        pltpu.SMEM((8,), jnp.int32),
        pltpu.SemaphoreType.DMA,
    ],
    compiler_params=pltpu.CompilerParams(
        use_tc_tiling_on_sc=True, needs_layout_passes=False
    ),
)
def _emb_lookup(idx_hbm, wt_hbm, out_hbm, xv, vbuf, pbuf, gbuf, dst, cnt, sem):
    wid = lax.axis_index("s") * _NC + lax.axis_index("c")
    base = wid * _BPW
    pltpu.sync_copy(idx_hbm.at[pl.ds(base, _BPW)], xv)
    iota = lax.iota(jnp.int32, _L)

    def chunk_body(c):
        cbase = c * _CHUNK

        for r in range(8):
            cnt[r] = 0

        def classify(g):
            vec = xv[pl.ds(cbase + g * _L, _L)]
            pos = g * _L + iota
            rr = (vec >> 4) & 7
            for r in range(8):
                mask = rr == r
                n = plsc.all_reduce_population_count(mask)[0]
                cur = cnt[r]
                plsc.store_compressed(vbuf.at[r, pl.ds(cur, _L)], vec, mask=mask)
                plsc.store_compressed(pbuf.at[r, pl.ds(cur, _L)], pos, mask=mask)
                cnt[r] = cur + n

        pl.loop(0, _CHUNK // _L)(classify)

        for r in range(8):
            vbuf[r, pl.ds(cnt[r], _L)] = jnp.full((_L,), r * _L, jnp.int32)
            pbuf[r, pl.ds(cnt[r], _L)] = jnp.full((_L,), _CHUNK, jnp.int32)

        for r in range(8):
            nrounds = (cnt[r] + 7) >> 3

            def round_body(blk, r=r):
                vec = vbuf[r, pl.ds(blk * 8, _L)]
                pos = pbuf[r, pl.ds(blk * 8, _L)]
                vb = (vec >> 7) << 7
                copies = [
                    pltpu.async_copy(
                        wt_hbm.at[
                            :, :, pl.ds(pl.multiple_of(vb[j], 128), 128)
                        ].at[:, :, pl.ds(r * _L, _L)],
                        gbuf.at[j, :, :, pl.ds(0, _L)],
                        sem,
                    )
                    for j in range(8)
                ]
                for cp in copies:
                    cp.wait()
                lane = vec & 15
                blk8 = iota & 7
                for d in range(DIM):
                    tr, s = d // 8, d % 8
                    vals = plsc.load_gather(
                        gbuf, [blk8, jnp.full((_L,), tr, jnp.int32),
                               jnp.full((_L,), s, jnp.int32), lane]
                    )
                    plsc.store_scatter(
                        dst, [jnp.full((_L,), d, jnp.int32), pos], vals,
                        mask=iota < 8,
                    )

            pl.loop(0, nrounds)(round_body)

        for tr in range(4):
            pltpu.sync_copy(
                dst.at[pl.ds(tr * 8, 8), pl.ds(0, _CHUNK)],
                out_hbm.at[pl.ds(tr * 8, 8), pl.ds(base + cbase, _CHUNK)],
            )

    pl.loop(0, _NCH)(chunk_body)


def kernel(x, weight):
    out_t = _emb_lookup(x.astype(jnp.int32), weight.T.reshape(4, 8, NUM_EMB))
    return out_t.T


# speedup vs baseline: 4.5894x; 1.1861x over previous
import functools

import jax
import jax.numpy as jnp
from jax import lax
from jax.experimental import pallas as pl
from jax.experimental.pallas import tpu as pltpu
from jax.experimental.pallas import tpu_sc as plsc

NUM_EMB = 1_000_000
DIM = 32
BATCH = 16384

_NC = 2
_NS = 16
_NW = _NC * _NS
_BPW = BATCH // _NW
_CHUNK = 128
_NCH = _BPW // _CHUNK
_L = 16
_BKT = _CHUNK + _L

_mesh = plsc.VectorSubcoreMesh(core_axis_name="c", subcore_axis_name="s")


@functools.partial(
    pl.kernel,
    mesh=_mesh,
    out_type=jax.ShapeDtypeStruct((DIM, BATCH), jnp.float32),
    scratch_types=[
        pltpu.VMEM((_BPW,), jnp.int32),
        pltpu.VMEM((8, _BKT), jnp.int32),
        pltpu.VMEM((8, _BKT), jnp.int32),
        pltpu.VMEM((2 * 8, 4, 8, 128), jnp.float32),
        pltpu.VMEM((DIM, _CHUNK + _L), jnp.float32),
        pltpu.SMEM((8,), jnp.int32),
        pltpu.SemaphoreType.DMA,
    ],
    compiler_params=pltpu.CompilerParams(
        use_tc_tiling_on_sc=True, needs_layout_passes=False
    ),
)
def _emb_lookup(idx_hbm, wt_hbm, out_hbm, xv, vbuf, pbuf, gbuf, dst, cnt, sem):
    wid = lax.axis_index("s") * _NC + lax.axis_index("c")
    base = wid * _BPW
    pltpu.sync_copy(idx_hbm.at[pl.ds(base, _BPW)], xv)
    iota = lax.iota(jnp.int32, _L)

    def chunk_body(c):
        cbase = c * _CHUNK

        for r in range(8):
            cnt[r] = 0

        def classify(g):
            vec = xv[pl.ds(cbase + g * _L, _L)]
            pos = g * _L + iota
            rr = (vec >> 4) & 7
            for r in range(8):
                mask = rr == r
                n = plsc.all_reduce_population_count(mask)[0]
                cur = cnt[r]
                plsc.store_compressed(vbuf.at[r, pl.ds(cur, _L)], vec, mask=mask)
                plsc.store_compressed(pbuf.at[r, pl.ds(cur, _L)], pos, mask=mask)
                cnt[r] = cur + n

        pl.loop(0, _CHUNK // _L)(classify)

        for r in range(8):
            vbuf[r, pl.ds(cnt[r], _L)] = jnp.full((_L,), r * _L, jnp.int32)
            pbuf[r, pl.ds(cnt[r], _L)] = jnp.full((_L,), _CHUNK, jnp.int32)

        for r in range(8):
            nrounds = (cnt[r] + 7) >> 3

            def round_body(blk, r=r, nrounds=nrounds):
                par = (blk & 1) << 3

                @pl.when(blk < nrounds)
                def _fire():
                    vec = vbuf[r, pl.ds(blk * 8, _L)]
                    vb = (vec >> 7) << 7
                    for j in range(8):
                        pltpu.async_copy(
                            wt_hbm.at[
                                :, :, pl.ds(pl.multiple_of(vb[j], 128), 128)
                            ].at[:, :, pl.ds(r * _L, _L)],
                            gbuf.at[par + j, :, :, pl.ds(0, _L)],
                            sem,
                        )

                @pl.when(blk > 0)
                def _drain_extract():
                    prev = par ^ 8
                    for j in range(8):
                        pltpu.make_async_copy(
                            wt_hbm.at[:, :, pl.ds(0, _L)],
                            gbuf.at[prev + j, :, :, pl.ds(0, _L)],
                            sem,
                        ).wait()
                    vecp = vbuf[r, pl.ds(blk * 8 - 8, _L)]
                    posp = pbuf[r, pl.ds(blk * 8 - 8, _L)]
                    lane = vecp & 15
                    blk8 = prev + (iota & 7)
                    for d in range(DIM):
                        tr, s = d // 8, d % 8
                        vals = plsc.load_gather(
                            gbuf, [blk8, jnp.full((_L,), tr, jnp.int32),
                                   jnp.full((_L,), s, jnp.int32), lane]
                        )
                        plsc.store_scatter(
                            dst, [jnp.full((_L,), d, jnp.int32), posp], vals,
                            mask=iota < 8,
                        )

            pl.loop(0, nrounds + 1)(round_body)

        for tr in range(4):
            pltpu.sync_copy(
                dst.at[pl.ds(tr * 8, 8), pl.ds(0, _CHUNK)],
                out_hbm.at[pl.ds(tr * 8, 8), pl.ds(base + cbase, _CHUNK)],
            )

    pl.loop(0, _NCH)(chunk_body)


def kernel(x, weight):
    out_t = _emb_lookup(x.astype(jnp.int32), weight.T.reshape(4, 8, NUM_EMB))
    return out_t.T
